# Initial kernel scaffold; baseline (speedup 1.0000x reference)
import jax
import jax.numpy as jnp
from jax.experimental import pallas as pl


def kernel(edge_index, W1, b1, W2, b2, Wf1, bf1, Wf2, bf2):
    raise NotImplementedError("write your pallas kernel here")


# speedup vs baseline: 9.2670x; 9.2670x over previous
import functools

import jax
import jax.numpy as jnp
from jax import lax
from jax.experimental import pallas as pl
from jax.experimental.pallas import tpu as pltpu
from jax.experimental.pallas import tpu_sc as plsc

N = 10000
NPAD = 10240
HID = 128
EMB = 256
E = 320000
NC = 2
NS = 16
EW = 128
ROWS_TOTAL = 2560
ROWS_PER_TILE = ROWS_TOTAL // (NC * NS)
ROWS_PER_SUB = ROWS_TOTAL // NS
E_PAD = ROWS_TOTAL * EW
NPT = NPAD // NS

_MESH = plsc.VectorSubcoreMesh(
    core_axis_name="c", subcore_axis_name="s", num_cores=NC, num_subcores=NS)


@functools.partial(
    pl.kernel,
    out_type=[jax.ShapeDtypeStruct((NPAD,), jnp.float32)] * 4,
    mesh=_MESH,
    scratch_types=[
        pltpu.VMEM_SHARED((NPAD,), jnp.float32),
        pltpu.VMEM_SHARED((NPAD,), jnp.float32),
        pltpu.VMEM_SHARED((NPAD,), jnp.float32),
        pltpu.VMEM_SHARED((NPAD,), jnp.float32),
        pltpu.VMEM((8, EW), jnp.int32),
        pltpu.VMEM((8, EW), jnp.int32),
        pltpu.VMEM((EW,), jnp.float32),
        pltpu.VMEM((EW,), jnp.float32),
        pltpu.VMEM((EW,), jnp.float32),
        pltpu.VMEM((NPT,), jnp.float32),
        pltpu.SemaphoreType.DMA,
    ],
)
def _sc_deg_agg1(src_hbm, dst_hbm, zeros_hbm,
                 a0_hbm, a1_hbm, b0_hbm, b1_hbm,
                 indeg_sp, outdeg_sp, a_sp, b_sp,
                 src_v, dst_v, va, vb, ones_v, zbuf, sem):
    c = lax.axis_index("c")
    s = lax.axis_index("s")
    wid = s * NC + c

    for i in range(EW // 16):
        ones_v[pl.ds(i * 16, 16)] = jnp.ones((16,), jnp.float32)

    pltpu.sync_copy(zeros_hbm, zbuf)
    zsl = pl.ds(s * NPT, NPT)
    pltpu.sync_copy(zbuf, indeg_sp.at[zsl])
    pltpu.sync_copy(zbuf, outdeg_sp.at[zsl])
    pltpu.sync_copy(zbuf, a_sp.at[zsl])
    pltpu.sync_copy(zbuf, b_sp.at[zsl])
    plsc.subcore_barrier()

    @pl.loop(0, ROWS_PER_SUB // 8)
    def _p1(batch):
        r = s * ROWS_PER_SUB + batch * 8
        pltpu.sync_copy(src_hbm.at[pl.ds(r, 8)], src_v)
        pltpu.sync_copy(dst_hbm.at[pl.ds(r, 8)], dst_v)
        for j in range(8):
            pltpu.sync_copy(ones_v, outdeg_sp.at[src_v.at[j]], add=True)
            pltpu.sync_copy(ones_v, indeg_sp.at[dst_v.at[j]], add=True)

    plsc.subcore_barrier()

    @pl.loop(0, ROWS_PER_TILE // 8)
    def _p2(batch):
        r = wid * ROWS_PER_TILE + batch * 8
        pltpu.sync_copy(src_hbm.at[pl.ds(r, 8)], src_v)
        pltpu.sync_copy(dst_hbm.at[pl.ds(r, 8)], dst_v)
        for j in range(8):
            pltpu.async_copy(indeg_sp.at[src_v.at[j]], va, sem).wait()
            pltpu.async_copy(outdeg_sp.at[src_v.at[j]], vb, sem).wait()
            pltpu.sync_copy(va, a_sp.at[dst_v.at[j]], add=True)
            pltpu.sync_copy(vb, b_sp.at[dst_v.at[j]], add=True)

    plsc.subcore_barrier()

    @pl.when(c == 0)
    def _():
        pltpu.sync_copy(a_sp.at[zsl], zbuf)
        pltpu.sync_copy(zbuf, a0_hbm.at[zsl])
        pltpu.sync_copy(b_sp.at[zsl], zbuf)
        pltpu.sync_copy(zbuf, b0_hbm.at[zsl])

    @pl.when(c == 1)
    def _():
        pltpu.sync_copy(a_sp.at[zsl], zbuf)
        pltpu.sync_copy(zbuf, a1_hbm.at[zsl])
        pltpu.sync_copy(b_sp.at[zsl], zbuf)
        pltpu.sync_copy(zbuf, b1_hbm.at[zsl])


@functools.partial(
    pl.kernel,
    out_type=[jax.ShapeDtypeStruct((NPAD, HID), jnp.float32)] * 2,
    mesh=_MESH,
    scratch_types=[
        pltpu.VMEM_SHARED((NPAD, HID), jnp.float32),
        pltpu.VMEM((8, EW), jnp.int32),
        pltpu.VMEM((8, EW), jnp.int32),
        pltpu.VMEM((EW, HID), jnp.float32),
        pltpu.SemaphoreType.DMA,
    ],
)
def _sc_agg2(h1_hbm, src_hbm, dst_hbm, zeros_hbm, o0_hbm, o1_hbm,
             acc_sp, src_v, dst_v, rows_v, sem):
    c = lax.axis_index("c")
    s = lax.axis_index("s")
    wid = s * NC + c

    pltpu.sync_copy(zeros_hbm, rows_v)
    for k in range(NPT // EW):
        pltpu.sync_copy(rows_v, acc_sp.at[pl.ds(s * NPT + k * EW, EW)])
    plsc.subcore_barrier()

    @pl.loop(0, ROWS_PER_TILE // 8)
    def _body(batch):
        r = wid * ROWS_PER_TILE + batch * 8
        pltpu.sync_copy(src_hbm.at[pl.ds(r, 8)], src_v)
        pltpu.sync_copy(dst_hbm.at[pl.ds(r, 8)], dst_v)
        for j in range(8):
            pltpu.async_copy(h1_hbm.at[src_v.at[j]], rows_v, sem).wait()
            pltpu.sync_copy(rows_v, acc_sp.at[dst_v.at[j]], add=True)

    plsc.subcore_barrier()

    for k in range(NPT // EW):
        sl = pl.ds(s * NPT + k * EW, EW)
        pltpu.sync_copy(acc_sp.at[sl], rows_v)

        @pl.when(c == 0)
        def _():
            pltpu.sync_copy(rows_v, o0_hbm.at[sl])

        @pl.when(c == 1)
        def _():
            pltpu.sync_copy(rows_v, o1_hbm.at[sl])


def _tc_h1_body(a0, a1, b0, b1, W1, bias, out):
    i = pl.program_id(0)
    av = a0[...] + a1[...]
    bv = b0[...] + b1[...]
    h = av * W1[0:1, :] + bv * W1[1:2, :] + bias[...]
    h = jnp.maximum(h, 0.0)
    rows = lax.broadcasted_iota(jnp.int32, (1024, 1), 0) + i * 1024
    out[...] = jnp.where(rows < N, h, 0.0)


def _tc_h1(a0, a1, b0, b1, W1, bias):
    col = pl.BlockSpec((1024, 1), lambda i: (i, 0))
    return pl.pallas_call(
        _tc_h1_body,
        grid=(NPAD // 1024,),
        in_specs=[col, col, col, col,
                  pl.BlockSpec((2, HID), lambda i: (0, 0)),
                  pl.BlockSpec((1, HID), lambda i: (0, 0))],
        out_specs=pl.BlockSpec((1024, HID), lambda i: (i, 0)),
        out_shape=jax.ShapeDtypeStruct((NPAD, HID), jnp.float32),
    )(a0, a1, b0, b1, W1, bias)


def _tc_head_body(o0, o1, W2, b2, Wf1, bf1, Wf2, bf2, ge, metric):
    i = pl.program_id(0)
    agg = o0[...] + o1[...]
    h2 = jnp.dot(agg, W2[...], preferred_element_type=jnp.float32) + b2[...]
    part = jnp.sum(jnp.maximum(h2, 0.0), axis=0, keepdims=True)

    @pl.when(i == 0)
    def _():
        ge[...] = part

    @pl.when(i > 0)
    def _():
        ge[...] += part

    @pl.when(i == pl.num_programs(0) - 1)
    def _():
        g = ge[...]
        hm = jnp.maximum(
            jnp.dot(g, Wf1[...], preferred_element_type=jnp.float32) + bf1[...], 0.0)
        metric[...] = jnp.dot(hm, Wf2[...],
                              preferred_element_type=jnp.float32) + bf2[...]


def _tc_head(o0, o1, W2, b2, Wf1, bf1, Wf2, bf2):
    blk = pl.BlockSpec((1000, HID), lambda i: (i, 0))
    full = lambda shape: pl.BlockSpec(shape, lambda i: (0, 0))
    return pl.pallas_call(
        _tc_head_body,
        grid=(N // 1000,),
        in_specs=[blk, blk, full((HID, EMB)), full((1, EMB)),
                  full((EMB, HID)), full((1, HID)), full((HID, 1)), full((1, 1))],
        out_specs=[full((1, EMB)), full((1, 1))],
        out_shape=[jax.ShapeDtypeStruct((1, EMB), jnp.float32),
                   jax.ShapeDtypeStruct((1, 1), jnp.float32)],
    )(o0, o1, W2, b2, Wf1, bf1, Wf2, bf2)


def kernel(edge_index, W1, b1, W2, b2, Wf1, bf1, Wf2, bf2):
    src = edge_index[0].astype(jnp.int32)
    dst = edge_index[1].astype(jnp.int32)
    pad = N + (jnp.arange(E_PAD - E, dtype=jnp.int32) % (NPAD - N))
    src_p = jnp.concatenate([src, pad]).reshape(ROWS_TOTAL, EW)
    dst_p = jnp.concatenate([dst, pad]).reshape(ROWS_TOTAL, EW)
    zeros1 = jnp.zeros((NPT,), jnp.float32)
    zeros2 = jnp.zeros((EW, HID), jnp.float32)

    a0, a1, b0, b1v = _sc_deg_agg1(src_p, dst_p, zeros1)
    h1 = _tc_h1(a0.reshape(NPAD, 1), a1.reshape(NPAD, 1),
                b0.reshape(NPAD, 1), b1v.reshape(NPAD, 1),
                W1, b1.reshape(1, HID))
    o0, o1 = _sc_agg2(h1, src_p, dst_p, zeros2)
    ge, metric = _tc_head(o0, o1, W2, b2.reshape(1, EMB),
                          Wf1, bf1.reshape(1, HID), Wf2, bf2.reshape(1, 1))
    return (ge, metric)


# speedup vs baseline: 13.6209x; 1.4698x over previous
import functools

import jax
import jax.numpy as jnp
from jax import lax
from jax.experimental import pallas as pl
from jax.experimental.pallas import tpu as pltpu
from jax.experimental.pallas import tpu_sc as plsc

N = 10000
NPAD = 10240
HID = 128
EMB = 256
E = 320000
NC = 2
NS = 16
EW = 128
ROWS_TOTAL = 2560
ROWS_PER_TILE = ROWS_TOTAL // (NC * NS)
ROWS_PER_SUB = ROWS_TOTAL // NS
E_PAD = ROWS_TOTAL * EW
NPT = NPAD // NS
NB = 5

_MESH = plsc.VectorSubcoreMesh(
    core_axis_name="c", subcore_axis_name="s", num_cores=NC, num_subcores=NS)


@functools.partial(
    pl.kernel,
    out_type=[jax.ShapeDtypeStruct((NPAD,), jnp.float32)] * 4,
    mesh=_MESH,
    compiler_params=pltpu.CompilerParams(use_tc_tiling_on_sc=False),
    scratch_types=[
        pltpu.VMEM_SHARED((NPAD,), jnp.float32),
        pltpu.VMEM_SHARED((NPAD,), jnp.float32),
        pltpu.VMEM_SHARED((NPAD,), jnp.float32),
        pltpu.VMEM_SHARED((NPAD,), jnp.float32),
        pltpu.VMEM((ROWS_PER_SUB, EW), jnp.int32),
        pltpu.VMEM((ROWS_PER_SUB, EW), jnp.int32),
        pltpu.VMEM((ROWS_PER_TILE, EW), jnp.int32),
        pltpu.VMEM((ROWS_PER_TILE, EW), jnp.int32),
        pltpu.VMEM((EW,), jnp.float32),
        [pltpu.VMEM((EW,), jnp.float32)] * NB,
        [pltpu.VMEM((EW,), jnp.float32)] * NB,
        pltpu.VMEM((NPT,), jnp.float32),
        [pltpu.SemaphoreType.DMA] * NB,
        [pltpu.SemaphoreType.DMA] * NB,
        [pltpu.SemaphoreType.DMA] * NB,
        [pltpu.SemaphoreType.DMA] * NB,
        pltpu.SemaphoreType.DMA,
    ],
)
def _sc_deg_agg1(src_hbm, dst_hbm, ones_hbm, zeros_hbm,
                 a0_hbm, a1_hbm, b0_hbm, b1_hbm,
                 indeg_sp, outdeg_sp, a_sp, b_sp,
                 src1_v, dst1_v, src2_v, dst2_v, ones_v, vas, vbs, zbuf,
                 gsems, ssems, gsems2, ssems2, psem):
    c = lax.axis_index("c")
    s = lax.axis_index("s")
    wid = s * NC + c

    pltpu.sync_copy(ones_hbm, ones_v)
    pltpu.sync_copy(zeros_hbm, zbuf)
    pltpu.sync_copy(src_hbm.at[pl.ds(s * ROWS_PER_SUB, ROWS_PER_SUB)], src1_v)
    pltpu.sync_copy(dst_hbm.at[pl.ds(s * ROWS_PER_SUB, ROWS_PER_SUB)], dst1_v)
    pltpu.sync_copy(src_hbm.at[pl.ds(wid * ROWS_PER_TILE, ROWS_PER_TILE)], src2_v)
    pltpu.sync_copy(dst_hbm.at[pl.ds(wid * ROWS_PER_TILE, ROWS_PER_TILE)], dst2_v)
    zsl = pl.ds(s * NPT, NPT)
    pltpu.sync_copy(zbuf, indeg_sp.at[zsl])
    pltpu.sync_copy(zbuf, outdeg_sp.at[zsl])
    pltpu.sync_copy(zbuf, a_sp.at[zsl])
    pltpu.sync_copy(zbuf, b_sp.at[zsl])
    plsc.subcore_barrier()

    @pl.loop(0, ROWS_PER_SUB, step=8)
    def _p1(r0):
        descs = []
        for i in range(8):
            descs.append(pltpu.async_copy(
                ones_v, outdeg_sp.at[src1_v.at[r0 + i]], psem, add=True))
            descs.append(pltpu.async_copy(
                ones_v, indeg_sp.at[dst1_v.at[r0 + i]], psem, add=True))
        for d in descs:
            d.wait()

    plsc.subcore_barrier()

    for b in range(NB):
        pltpu.async_copy(indeg_sp.at[src2_v.at[b]], vas[b], gsems[b])
        pltpu.async_copy(outdeg_sp.at[src2_v.at[b]], vbs[b], gsems2[b])

    @pl.loop(0, ROWS_PER_TILE - NB, step=NB)
    def _p2(j0):
        scs = []
        for b in range(NB):
            pltpu.make_async_copy(indeg_sp.at[src2_v.at[j0 + b]], vas[b],
                                  gsems[b]).wait()
            pltpu.make_async_copy(outdeg_sp.at[src2_v.at[j0 + b]], vbs[b],
                                  gsems2[b]).wait()
            scs.append(pltpu.async_copy(vas[b], a_sp.at[dst2_v.at[j0 + b]],
                                        ssems[b], add=True))
            scs.append(pltpu.async_copy(vbs[b], b_sp.at[dst2_v.at[j0 + b]],
                                        ssems2[b], add=True))
        for b in range(NB):
            scs[2 * b].wait()
            scs[2 * b + 1].wait()
            pltpu.async_copy(indeg_sp.at[src2_v.at[j0 + NB + b]], vas[b],
                             gsems[b])
            pltpu.async_copy(outdeg_sp.at[src2_v.at[j0 + NB + b]], vbs[b],
                             gsems2[b])

    last = []
    for b in range(NB):
        j = ROWS_PER_TILE - NB + b
        pltpu.make_async_copy(indeg_sp.at[src2_v.at[j]], vas[b], gsems[b]).wait()
        pltpu.make_async_copy(outdeg_sp.at[src2_v.at[j]], vbs[b], gsems2[b]).wait()
        last.append(pltpu.async_copy(vas[b], a_sp.at[dst2_v.at[j]],
                                     ssems[b], add=True))
        last.append(pltpu.async_copy(vbs[b], b_sp.at[dst2_v.at[j]],
                                     ssems2[b], add=True))
    for d in last:
        d.wait()
    plsc.subcore_barrier()

    @pl.when(c == 0)
    def _():
        pltpu.sync_copy(a_sp.at[zsl], zbuf)
        pltpu.sync_copy(zbuf, a0_hbm.at[zsl])
        pltpu.sync_copy(b_sp.at[zsl], zbuf)
        pltpu.sync_copy(zbuf, b0_hbm.at[zsl])

    @pl.when(c == 1)
    def _():
        pltpu.sync_copy(a_sp.at[zsl], zbuf)
        pltpu.sync_copy(zbuf, a1_hbm.at[zsl])
        pltpu.sync_copy(b_sp.at[zsl], zbuf)
        pltpu.sync_copy(zbuf, b1_hbm.at[zsl])


@functools.partial(
    pl.kernel,
    out_type=[jax.ShapeDtypeStruct((NPAD, HID // 2), jnp.float32)] * 4,
    mesh=_MESH,
    compiler_params=pltpu.CompilerParams(use_tc_tiling_on_sc=False),
    scratch_types=[
        pltpu.VMEM_SHARED((NPAD, HID // 2), jnp.float32),
        [pltpu.SemaphoreType.DMA] * NB,
        [pltpu.SemaphoreType.DMA] * NB,
    ],
)
def _sc_agg2(h1a_hbm, h1b_hbm, src_hbm, dst_hbm, zeros_hbm,
             o0a_hbm, o1a_hbm, o0b_hbm, o1b_hbm, acc_sp, gsems, ssems):
    pl.run_scoped(
        functools.partial(_sc_agg2_body, h1a_hbm, h1b_hbm, src_hbm, dst_hbm,
                          zeros_hbm, o0a_hbm, o1a_hbm, o0b_hbm, o1b_hbm,
                          acc_sp, gsems, ssems),
        pltpu.VMEM((ROWS_PER_TILE, EW), jnp.int32),
        pltpu.VMEM((ROWS_PER_TILE, EW), jnp.int32),
        *([pltpu.VMEM((EW, HID // 2), jnp.float32)] * NB),
    )


def _sc_agg2_body(h1a_hbm, h1b_hbm, src_hbm, dst_hbm, zeros_hbm,
                  o0a_hbm, o1a_hbm, o0b_hbm, o1b_hbm,
                  acc_sp, gsems, ssems, src_v, dst_v, *rows):
    c = lax.axis_index("c")
    s = lax.axis_index("s")
    wid = s * NC + c

    pltpu.sync_copy(src_hbm.at[pl.ds(wid * ROWS_PER_TILE, ROWS_PER_TILE)], src_v)
    pltpu.sync_copy(dst_hbm.at[pl.ds(wid * ROWS_PER_TILE, ROWS_PER_TILE)], dst_v)

    for h_hbm, oc0_hbm, oc1_hbm in ((h1a_hbm, o0a_hbm, o1a_hbm),
                                    (h1b_hbm, o0b_hbm, o1b_hbm)):
        pltpu.sync_copy(zeros_hbm, rows[0])
        zds = []
        for k in range(NPT // EW):
            zds.append(pltpu.async_copy(
                rows[0], acc_sp.at[pl.ds(s * NPT + k * EW, EW)], ssems[0]))
        for d in zds:
            d.wait()
        plsc.subcore_barrier()

        for b in range(NB):
            pltpu.async_copy(h_hbm.at[src_v.at[b]], rows[b], gsems[b])

        @pl.loop(0, ROWS_PER_TILE - NB, step=NB)
        def _body(j0):
            scs = []
            for b in range(NB):
                pltpu.make_async_copy(h_hbm.at[src_v.at[j0 + b]], rows[b],
                                      gsems[b]).wait()
                scs.append(pltpu.async_copy(rows[b], acc_sp.at[dst_v.at[j0 + b]],
                                            ssems[b], add=True))
            for b in range(NB):
                scs[b].wait()
                pltpu.async_copy(h_hbm.at[src_v.at[j0 + NB + b]], rows[b],
                                 gsems[b])

        last = []
        for b in range(NB):
            j = ROWS_PER_TILE - NB + b
            pltpu.make_async_copy(h_hbm.at[src_v.at[j]], rows[b], gsems[b]).wait()
            last.append(pltpu.async_copy(rows[b], acc_sp.at[dst_v.at[j]],
                                         ssems[b], add=True))
        for d in last:
            d.wait()
        plsc.subcore_barrier()

        rds = []
        for k in range(min(NB, NPT // EW)):
            sl = pl.ds(s * NPT + k * EW, EW)
            rds.append(pltpu.async_copy(acc_sp.at[sl], rows[k], gsems[k]))
        for k in range(NPT // EW):
            sl = pl.ds(s * NPT + k * EW, EW)
            if k >= NB:
                rds.append(pltpu.async_copy(acc_sp.at[sl], rows[k % NB],
                                            gsems[k % NB]))
            rds[k].wait()

            @pl.when(c == 0)
            def _():
                pltpu.sync_copy(rows[k % NB], oc0_hbm.at[sl])

            @pl.when(c == 1)
            def _():
                pltpu.sync_copy(rows[k % NB], oc1_hbm.at[sl])
        plsc.subcore_barrier()


def _tc_h1_body(a0, a1, b0, b1, W1, bias, outa, outb):
    i = pl.program_id(0)
    av = a0[...] + a1[...]
    bv = b0[...] + b1[...]
    h = av * W1[0:1, :] + bv * W1[1:2, :] + bias[...]
    h = jnp.maximum(h, 0.0)
    rows = lax.broadcasted_iota(jnp.int32, (1024, 1), 0) + i * 1024
    h = jnp.where(rows < N, h, 0.0)
    outa[...] = h[:, :HID // 2]
    outb[...] = h[:, HID // 2:]


def _tc_h1(a0, a1, b0, b1, W1, bias):
    col = pl.BlockSpec((1024, 1), lambda i: (i, 0))
    half = pl.BlockSpec((1024, HID // 2), lambda i: (i, 0))
    return pl.pallas_call(
        _tc_h1_body,
        grid=(NPAD // 1024,),
        in_specs=[col, col, col, col,
                  pl.BlockSpec((2, HID), lambda i: (0, 0)),
                  pl.BlockSpec((1, HID), lambda i: (0, 0))],
        out_specs=[half, half],
        out_shape=[jax.ShapeDtypeStruct((NPAD, HID // 2), jnp.float32)] * 2,
    )(a0, a1, b0, b1, W1, bias)


def _tc_head_body(o0a, o1a, o0b, o1b, W2, b2, Wf1, bf1, Wf2, bf2, ge, metric):
    i = pl.program_id(0)
    aggA = o0a[...] + o1a[...]
    aggB = o0b[...] + o1b[...]
    h2 = (jnp.dot(aggA, W2[...][:HID // 2, :],
                  preferred_element_type=jnp.float32)
          + jnp.dot(aggB, W2[...][HID // 2:, :],
                    preferred_element_type=jnp.float32) + b2[...])
    part = jnp.sum(jnp.maximum(h2, 0.0), axis=0, keepdims=True)

    @pl.when(i == 0)
    def _():
        ge[...] = part

    @pl.when(i > 0)
    def _():
        ge[...] += part

    @pl.when(i == pl.num_programs(0) - 1)
    def _():
        g = ge[...]
        hm = jnp.maximum(
            jnp.dot(g, Wf1[...], preferred_element_type=jnp.float32) + bf1[...], 0.0)
        metric[...] = jnp.dot(hm, Wf2[...],
                              preferred_element_type=jnp.float32) + bf2[...]


def _tc_head(o0a, o1a, o0b, o1b, W2, b2, Wf1, bf1, Wf2, bf2):
    blk = pl.BlockSpec((1000, HID // 2), lambda i: (i, 0))
    full = lambda shape: pl.BlockSpec(shape, lambda i: (0, 0))
    return pl.pallas_call(
        _tc_head_body,
        grid=(N // 1000,),
        in_specs=[blk, blk, blk, blk, full((HID, EMB)), full((1, EMB)),
                  full((EMB, HID)), full((1, HID)), full((HID, 1)), full((1, 1))],
        out_specs=[full((1, EMB)), full((1, 1))],
        out_shape=[jax.ShapeDtypeStruct((1, EMB), jnp.float32),
                   jax.ShapeDtypeStruct((1, 1), jnp.float32)],
    )(o0a, o1a, o0b, o1b, W2, b2, Wf1, bf1, Wf2, bf2)


def kernel(edge_index, W1, b1, W2, b2, Wf1, bf1, Wf2, bf2):
    src = edge_index[0].astype(jnp.int32)
    dst = edge_index[1].astype(jnp.int32)
    pad = N + (jnp.arange(E_PAD - E, dtype=jnp.int32) % (NPAD - N))
    src_p = jnp.concatenate([src, pad]).reshape(ROWS_TOTAL, EW)
    dst_p = jnp.concatenate([dst, pad]).reshape(ROWS_TOTAL, EW)
    ones1 = jnp.ones((EW,), jnp.float32)
    zeros1 = jnp.zeros((NPT,), jnp.float32)
    zeros2 = jnp.zeros((EW, HID // 2), jnp.float32)

    a0, a1, b0v, b1v = _sc_deg_agg1(src_p, dst_p, ones1, zeros1)
    h1a, h1b = _tc_h1(a0.reshape(NPAD, 1), a1.reshape(NPAD, 1),
                      b0v.reshape(NPAD, 1), b1v.reshape(NPAD, 1),
                      W1, b1.reshape(1, HID))
    o0a, o1a, o0b, o1b = _sc_agg2(h1a, h1b, src_p, dst_p, zeros2)
    ge, metric = _tc_head(o0a, o1a, o0b, o1b, W2, b2.reshape(1, EMB),
                          Wf1, bf1.reshape(1, HID), Wf2, bf2.reshape(1, 1))
    return (ge, metric)


# speedup vs baseline: 14.4772x; 1.0629x over previous
import functools

import jax
import jax.numpy as jnp
from jax import lax
from jax.experimental import pallas as pl
from jax.experimental.pallas import tpu as pltpu
from jax.experimental.pallas import tpu_sc as plsc

N = 10000
NPAD = 10240
HID = 128
EMB = 256
E = 320000
NC = 2
NS = 16
EW = 128
ROWS_TOTAL = 2560
ROWS_PER_TILE = ROWS_TOTAL // (NC * NS)
ROWS_PER_SUB = ROWS_TOTAL // NS
E_PAD = ROWS_TOTAL * EW
NPT = NPAD // NS
NB = 5

_MESH = plsc.VectorSubcoreMesh(
    core_axis_name="c", subcore_axis_name="s", num_cores=NC, num_subcores=NS)


@functools.partial(
    pl.kernel,
    out_type=jax.ShapeDtypeStruct((4, NPAD), jnp.float32),
    mesh=_MESH,
    compiler_params=pltpu.CompilerParams(use_tc_tiling_on_sc=False),
    scratch_types=[
        pltpu.VMEM_SHARED((NPAD,), jnp.float32),
        pltpu.VMEM_SHARED((NPAD,), jnp.float32),
        pltpu.VMEM_SHARED((NPAD,), jnp.float32),
        pltpu.VMEM_SHARED((NPAD,), jnp.float32),
        pltpu.VMEM((ROWS_PER_SUB, EW), jnp.int32),
        pltpu.VMEM((ROWS_PER_SUB, EW), jnp.int32),
        pltpu.VMEM((ROWS_PER_TILE, EW), jnp.int32),
        pltpu.VMEM((ROWS_PER_TILE, EW), jnp.int32),
        pltpu.VMEM((EW,), jnp.float32),
        [pltpu.VMEM((EW,), jnp.float32)] * NB,
        [pltpu.VMEM((EW,), jnp.float32)] * NB,
        pltpu.VMEM((NPT,), jnp.float32),
        [pltpu.SemaphoreType.DMA] * NB,
        [pltpu.SemaphoreType.DMA] * NB,
        [pltpu.SemaphoreType.DMA] * NB,
        [pltpu.SemaphoreType.DMA] * NB,
        pltpu.SemaphoreType.DMA,
    ],
)
def _sc_deg_agg1(src_hbm, dst_hbm, ones_hbm, zeros_hbm, p_hbm,
                 indeg_sp, outdeg_sp, a_sp, b_sp,
                 src1_v, dst1_v, src2_v, dst2_v, ones_v, vas, vbs, zbuf,
                 gsems, ssems, gsems2, ssems2, psem):
    c = lax.axis_index("c")
    s = lax.axis_index("s")
    wid = s * NC + c

    pltpu.sync_copy(ones_hbm, ones_v)
    pltpu.sync_copy(zeros_hbm, zbuf)
    pltpu.sync_copy(src_hbm.at[pl.ds(s * ROWS_PER_SUB, ROWS_PER_SUB)], src1_v)
    pltpu.sync_copy(dst_hbm.at[pl.ds(s * ROWS_PER_SUB, ROWS_PER_SUB)], dst1_v)
    pltpu.sync_copy(src_hbm.at[pl.ds(wid * ROWS_PER_TILE, ROWS_PER_TILE)], src2_v)
    pltpu.sync_copy(dst_hbm.at[pl.ds(wid * ROWS_PER_TILE, ROWS_PER_TILE)], dst2_v)
    zsl = pl.ds(s * NPT, NPT)
    pltpu.sync_copy(zbuf, indeg_sp.at[zsl])
    pltpu.sync_copy(zbuf, outdeg_sp.at[zsl])
    pltpu.sync_copy(zbuf, a_sp.at[zsl])
    pltpu.sync_copy(zbuf, b_sp.at[zsl])
    plsc.subcore_barrier()

    @pl.loop(0, ROWS_PER_SUB, step=8)
    def _p1(r0):
        descs = []
        for i in range(8):
            descs.append(pltpu.async_copy(
                ones_v, outdeg_sp.at[src1_v.at[r0 + i]], psem, add=True))
            descs.append(pltpu.async_copy(
                ones_v, indeg_sp.at[dst1_v.at[r0 + i]], psem, add=True))
        for d in descs:
            d.wait()

    plsc.subcore_barrier()

    for b in range(NB):
        pltpu.async_copy(indeg_sp.at[src2_v.at[b]], vas[b], gsems[b])
        pltpu.async_copy(outdeg_sp.at[src2_v.at[b]], vbs[b], gsems2[b])

    @pl.loop(0, ROWS_PER_TILE - NB, step=NB)
    def _p2(j0):
        scs = []
        for b in range(NB):
            pltpu.make_async_copy(indeg_sp.at[src2_v.at[j0 + b]], vas[b],
                                  gsems[b]).wait()
            pltpu.make_async_copy(outdeg_sp.at[src2_v.at[j0 + b]], vbs[b],
                                  gsems2[b]).wait()
            scs.append(pltpu.async_copy(vas[b], a_sp.at[dst2_v.at[j0 + b]],
                                        ssems[b], add=True))
            scs.append(pltpu.async_copy(vbs[b], b_sp.at[dst2_v.at[j0 + b]],
                                        ssems2[b], add=True))
        for b in range(NB):
            scs[2 * b].wait()
            scs[2 * b + 1].wait()
            pltpu.async_copy(indeg_sp.at[src2_v.at[j0 + NB + b]], vas[b],
                             gsems[b])
            pltpu.async_copy(outdeg_sp.at[src2_v.at[j0 + NB + b]], vbs[b],
                             gsems2[b])

    last = []
    for b in range(NB):
        j = ROWS_PER_TILE - NB + b
        pltpu.make_async_copy(indeg_sp.at[src2_v.at[j]], vas[b], gsems[b]).wait()
        pltpu.make_async_copy(outdeg_sp.at[src2_v.at[j]], vbs[b], gsems2[b]).wait()
        last.append(pltpu.async_copy(vas[b], a_sp.at[dst2_v.at[j]],
                                     ssems[b], add=True))
        last.append(pltpu.async_copy(vbs[b], b_sp.at[dst2_v.at[j]],
                                     ssems2[b], add=True))
    for d in last:
        d.wait()
    plsc.subcore_barrier()

    @pl.when(c == 0)
    def _():
        pltpu.sync_copy(a_sp.at[zsl], zbuf)
        pltpu.sync_copy(zbuf, p_hbm.at[0, zsl])
        pltpu.sync_copy(b_sp.at[zsl], zbuf)
        pltpu.sync_copy(zbuf, p_hbm.at[2, zsl])

    @pl.when(c == 1)
    def _():
        pltpu.sync_copy(a_sp.at[zsl], zbuf)
        pltpu.sync_copy(zbuf, p_hbm.at[1, zsl])
        pltpu.sync_copy(b_sp.at[zsl], zbuf)
        pltpu.sync_copy(zbuf, p_hbm.at[3, zsl])


@functools.partial(
    pl.kernel,
    out_type=jax.ShapeDtypeStruct((4, NPAD, HID // 2), jnp.float32),
    mesh=_MESH,
    compiler_params=pltpu.CompilerParams(use_tc_tiling_on_sc=False),
    scratch_types=[
        pltpu.VMEM_SHARED((NPAD, HID // 2), jnp.float32),
        [pltpu.SemaphoreType.DMA] * NB,
        [pltpu.SemaphoreType.DMA] * NB,
    ],
)
def _sc_agg2(h1a_hbm, h1b_hbm, src_hbm, dst_hbm, zeros_hbm, o_hbm,
             acc_sp, gsems, ssems):
    pl.run_scoped(
        functools.partial(_sc_agg2_body, h1a_hbm, h1b_hbm, src_hbm, dst_hbm,
                          zeros_hbm, o_hbm, acc_sp, gsems, ssems),
        pltpu.VMEM((ROWS_PER_TILE, EW), jnp.int32),
        pltpu.VMEM((ROWS_PER_TILE, EW), jnp.int32),
        *([pltpu.VMEM((EW, HID // 2), jnp.float32)] * NB),
    )


def _sc_agg2_body(h1a_hbm, h1b_hbm, src_hbm, dst_hbm, zeros_hbm, o_hbm,
                  acc_sp, gsems, ssems, src_v, dst_v, *rows):
    c = lax.axis_index("c")
    s = lax.axis_index("s")
    wid = s * NC + c

    pltpu.sync_copy(src_hbm.at[pl.ds(wid * ROWS_PER_TILE, ROWS_PER_TILE)], src_v)
    pltpu.sync_copy(dst_hbm.at[pl.ds(wid * ROWS_PER_TILE, ROWS_PER_TILE)], dst_v)

    for hi, h_hbm in enumerate((h1a_hbm, h1b_hbm)):
        pltpu.sync_copy(zeros_hbm, rows[0])
        zds = []
        for k in range(NPT // EW):
            zds.append(pltpu.async_copy(
                rows[0], acc_sp.at[pl.ds(s * NPT + k * EW, EW)], ssems[0]))
        for d in zds:
            d.wait()
        plsc.subcore_barrier()

        for b in range(NB):
            pltpu.async_copy(h_hbm.at[src_v.at[b]], rows[b], gsems[b])

        @pl.loop(0, ROWS_PER_TILE - NB, step=NB)
        def _body(j0):
            scs = []
            for b in range(NB):
                pltpu.make_async_copy(h_hbm.at[src_v.at[j0 + b]], rows[b],
                                      gsems[b]).wait()
                scs.append(pltpu.async_copy(rows[b], acc_sp.at[dst_v.at[j0 + b]],
                                            ssems[b], add=True))
            for b in range(NB):
                scs[b].wait()
                pltpu.async_copy(h_hbm.at[src_v.at[j0 + NB + b]], rows[b],
                                 gsems[b])

        last = []
        for b in range(NB):
            j = ROWS_PER_TILE - NB + b
            pltpu.make_async_copy(h_hbm.at[src_v.at[j]], rows[b], gsems[b]).wait()
            last.append(pltpu.async_copy(rows[b], acc_sp.at[dst_v.at[j]],
                                         ssems[b], add=True))
        for d in last:
            d.wait()
        plsc.subcore_barrier()

        rds = []
        for k in range(min(NB, NPT // EW)):
            sl = pl.ds(s * NPT + k * EW, EW)
            rds.append(pltpu.async_copy(acc_sp.at[sl], rows[k], gsems[k]))
        for k in range(NPT // EW):
            sl = pl.ds(s * NPT + k * EW, EW)
            if k >= NB:
                rds.append(pltpu.async_copy(acc_sp.at[sl], rows[k % NB],
                                            gsems[k % NB]))
            rds[k].wait()

            @pl.when(c == 0)
            def _():
                pltpu.sync_copy(rows[k % NB], o_hbm.at[2 * hi, sl])

            @pl.when(c == 1)
            def _():
                pltpu.sync_copy(rows[k % NB], o_hbm.at[2 * hi + 1, sl])
        plsc.subcore_barrier()


def _tc_h1_body(x, wcat, bias, outa, outb):
    h = lax.dot_general(x[...], wcat[...], (((0,), (0,)), ((), ())),
                        preferred_element_type=jnp.float32,
                        precision=lax.Precision.HIGHEST)
    h = jnp.maximum(h + bias[...], 0.0)
    rows = lax.broadcasted_iota(jnp.int32, (NPAD, 1), 0)
    h = jnp.where(rows < N, h, 0.0)
    outa[...] = h[:, :HID // 2]
    outb[...] = h[:, HID // 2:]


def _tc_h1(x, wcat, bias):
    full = lambda shape: pl.BlockSpec(shape, lambda: (0,) * len(shape))
    return pl.pallas_call(
        _tc_h1_body,
        in_specs=[full((4, NPAD)), full((4, HID)), full((1, HID))],
        out_specs=[full((NPAD, HID // 2))] * 2,
        out_shape=[jax.ShapeDtypeStruct((NPAD, HID // 2), jnp.float32)] * 2,
    )(x, wcat, bias)


def _tc_head_body(o, W2, b2, Wf1, bf1, Wf2, bf2, ge, metric):
    aggA = o[0] + o[1]
    aggB = o[2] + o[3]
    h2 = (jnp.dot(aggA, W2[...][:HID // 2, :],
                  preferred_element_type=jnp.float32)
          + jnp.dot(aggB, W2[...][HID // 2:, :],
                    preferred_element_type=jnp.float32) + b2[...])
    rows = lax.broadcasted_iota(jnp.int32, (NPAD, 1), 0)
    h2 = jnp.where(rows < N, jnp.maximum(h2, 0.0), 0.0)
    g = jnp.sum(h2, axis=0, keepdims=True)
    ge[...] = g
    hm = jnp.maximum(
        jnp.dot(g, Wf1[...], preferred_element_type=jnp.float32) + bf1[...], 0.0)
    metric[...] = jnp.dot(hm, Wf2[...],
                          preferred_element_type=jnp.float32) + bf2[...]


def _tc_head(o, W2, b2, Wf1, bf1, Wf2, bf2):
    full = lambda shape: pl.BlockSpec(shape, lambda: (0,) * len(shape))
    return pl.pallas_call(
        _tc_head_body,
        in_specs=[full((4, NPAD, HID // 2)), full((HID, EMB)), full((1, EMB)),
                  full((EMB, HID)), full((1, HID)), full((HID, 1)), full((1, 1))],
        out_specs=[full((1, EMB)), full((1, 1))],
        out_shape=[jax.ShapeDtypeStruct((1, EMB), jnp.float32),
                   jax.ShapeDtypeStruct((1, 1), jnp.float32)],
    )(o, W2, b2, Wf1, bf1, Wf2, bf2)


def kernel(edge_index, W1, b1, W2, b2, Wf1, bf1, Wf2, bf2):
    src = edge_index[0].astype(jnp.int32)
    dst = edge_index[1].astype(jnp.int32)
    pad = N + (jnp.arange(E_PAD - E, dtype=jnp.int32) % (NPAD - N))
    src_p = jnp.concatenate([src, pad]).reshape(ROWS_TOTAL, EW)
    dst_p = jnp.concatenate([dst, pad]).reshape(ROWS_TOTAL, EW)
    ones1 = jnp.ones((EW,), jnp.float32)
    zeros1 = jnp.zeros((NPT,), jnp.float32)
    zeros2 = jnp.zeros((EW, HID // 2), jnp.float32)

    p = _sc_deg_agg1(src_p, dst_p, ones1, zeros1)
    wcat = jnp.repeat(W1, 2, axis=0)
    h1a, h1b = _tc_h1(p, wcat, b1.reshape(1, HID))
    o = _sc_agg2(h1a, h1b, src_p, dst_p, zeros2)
    ge, metric = _tc_head(o, W2, b2.reshape(1, EMB),
                          Wf1, bf1.reshape(1, HID), Wf2, bf2.reshape(1, 1))
    return (ge, metric)
